# baseline (speedup 1.0000x reference)
import functools

import jax
import jax.numpy as jnp
from jax import lax
from jax.experimental import pallas as pl
from jax.experimental.pallas import tpu as pltpu
from jax.experimental.pallas import tpu_sc as plsc

N = 10000
E = 320000
FEAT = 128
B = 64
C = 10

NC = 2
NS = 16
NW = NC * NS
L = 16

_MESH = dict(core_axis_name="c", subcore_axis_name="s", num_cores=NC,
             num_subcores=NS)


def _wid():
    return lax.axis_index("s") * NC + lax.axis_index("c")


def _permute(v, idx):
    dnums = lax.GatherDimensionNumbers(
        offset_dims=(), collapsed_slice_dims=(0,), start_index_map=(0,))
    return lax.gather(v, idx.reshape(L, 1), dnums, (1,),
                      mode=lax.GatherScatterMode.PROMISE_IN_BOUNDS)


_EPW = E // NW
_EC = 80
_NCHUNK = _EPW // _EC
_STRIPE = (N // NS) // 8 * 8
_REM = N - (NS - 1) * _STRIPE - _STRIPE


def _conv_kernel_body(table, src, dst, out, agg, srcv, dstv, rows0, rows1,
                      zb, sem0, sem1, ssem0, ssem1):
    core = lax.axis_index("c")
    sid = lax.axis_index("s")
    wid = sid * NC + core

    ebase = wid * _EPW
    pltpu.sync_copy(src.at[pl.ds(ebase, _EPW)], srcv)
    pltpu.sync_copy(dst.at[pl.ds(ebase, _EPW)], dstv)

    bufs = (rows0, rows1)
    sems = (sem0, sem1)
    ssems = (ssem0, ssem1)

    def _start_g(q, b):
        pltpu.async_copy(table.at[srcv.at[pl.ds(q * _EC, _EC)]], bufs[b],
                         sems[b])

    def _wait_g(q, b):
        pltpu.make_async_copy(table.at[srcv.at[pl.ds(q * _EC, _EC)]],
                              bufs[b], sems[b]).wait()

    def _start_s(q, b):
        pltpu.async_copy(bufs[b], agg.at[dstv.at[pl.ds(q * _EC, _EC)]],
                         ssems[b], add=True)

    def _wait_s(q, b):
        pltpu.make_async_copy(bufs[b],
                              agg.at[dstv.at[pl.ds(q * _EC, _EC)]],
                              ssems[b]).wait()

    _start_g(0, 0)
    _start_g(1, 1)

    for r in range(L):
        for cth in range(FEAT // L):
            zb[r, pl.ds(cth * L, L)] = jnp.zeros((L,), jnp.float32)

    zbase = sid * _STRIPE

    def _zero(i, _):
        pltpu.sync_copy(zb, agg.at[pl.ds(zbase + i * L, L), :])
        return 0

    nz = _STRIPE // L + jnp.where(sid == NS - 1, _REM // L, 0)
    lax.fori_loop(0, nz, _zero, 0)
    plsc.subcore_barrier()

    def _pair(i, _):
        q0 = 2 * i
        _wait_g(q0, 0)
        _start_s(q0, 0)
        _wait_g(q0 + 1, 1)
        _start_s(q0 + 1, 1)
        _wait_s(q0, 0)

        @pl.when(q0 + 2 < _NCHUNK)
        def _g0():
            _start_g(q0 + 2, 0)

        _wait_s(q0 + 1, 1)

        @pl.when(q0 + 3 < _NCHUNK)
        def _g1():
            _start_g(q0 + 3, 1)

        return 0

    lax.fori_loop(0, _NCHUNK // 2, _pair, 0)
    _wait_g(_NCHUNK - 1, 0)
    _start_s(_NCHUNK - 1, 0)
    _wait_s(_NCHUNK - 1, 0)
    plsc.subcore_barrier()

    obase = sid * _STRIPE
    pltpu.sync_copy(agg.at[pl.ds(obase, _STRIPE), :],
                    out.at[core, pl.ds(obase, _STRIPE), :])

    @pl.when(sid == NS - 1)
    def _tail():
        tbase = NS * _STRIPE
        pltpu.sync_copy(agg.at[pl.ds(tbase, _REM), :],
                        out.at[core, pl.ds(tbase, _REM), :])


def _sc_conv(table, src, dst):
    mesh = plsc.VectorSubcoreMesh(**_MESH)

    fn = pl.kernel(
        _conv_kernel_body,
        out_type=jax.ShapeDtypeStruct((NC, N, FEAT), jnp.float32),
        mesh=mesh,
        compiler_params=pltpu.CompilerParams(needs_layout_passes=False),
        scratch_types=[
            pltpu.VMEM_SHARED((N, FEAT), jnp.float32),
            pltpu.VMEM((_EPW,), jnp.int32),
            pltpu.VMEM((_EPW,), jnp.int32),
            pltpu.VMEM((_EC, FEAT), jnp.float32),
            pltpu.VMEM((_EC, FEAT), jnp.float32),
            pltpu.VMEM((L, FEAT), jnp.float32),
            pltpu.SemaphoreType.DMA,
            pltpu.SemaphoreType.DMA,
            pltpu.SemaphoreType.DMA,
            pltpu.SemaphoreType.DMA,
        ],
    )
    return fn(table, src, dst)


def _seg_bounds(b_v, g0, g1):
    zero = jnp.zeros((L,), jnp.int32)

    def body(i, carry):
        lt0, eq0, lt1, eq1 = carry
        ch = b_v[pl.ds(i * L, L)]
        lt0 = lt0 + plsc.all_reduce_population_count(ch < g0)
        eq0 = eq0 + plsc.all_reduce_population_count(ch == g0)
        lt1 = lt1 + plsc.all_reduce_population_count(ch < g1)
        eq1 = eq1 + plsc.all_reduce_population_count(ch == g1)
        return lt0, eq0, lt1, eq1

    lt0, eq0, lt1, eq1 = lax.fori_loop(0, N // L, body,
                                       (zero, zero, zero, zero))
    return (jnp.max(lt0), jnp.max(eq0), jnp.max(lt1), jnp.max(eq1))


_PW = 128


def _pool_body(write_hp, table, wmul, mask, batch, mx, mn, hp, b_v, w_v, m_v,
               win, smx, smn):
    core = lax.axis_index("c")
    sid = lax.axis_index("s")
    wid = core * NS + sid
    g0 = wid * 2
    g1 = g0 + 1
    pltpu.sync_copy(batch, b_v.at[pl.ds(0, N)])
    pltpu.sync_copy(wmul, w_v.at[pl.ds(0, N)])
    pltpu.sync_copy(mask, m_v.at[pl.ds(0, N)])
    start0, n0, lt1, n1 = _seg_bounds(b_v, g0, g1)
    start1 = lt1

    nf = FEAT // L
    neg = jnp.full((L,), -jnp.inf, jnp.float32)
    zf = jnp.zeros((L,), jnp.float32)

    for g, start, n in ((g0, start0, n0), (g1, start1, n1)):
        astart = start // 8 * 8
        nwin = (start - astart + n + _PW - 1) // _PW

        def win_body(w, carry):
            accs = carry
            a = astart + w * _PW
            base = jnp.minimum(a, N - _PW)
            pltpu.sync_copy(table.at[pl.ds(base, _PW), :], win)

            if write_hp:
                def hp_row(r, _):
                    wm = w_v[pl.ds(base + r, L)][0]
                    for cth in range(nf):
                        v = win[r, pl.ds(cth * L, L)]
                        win[r, pl.ds(cth * L, L)] = v * wm
                    return 0

                lax.fori_loop(0, _PW, hp_row, 0)
                pltpu.sync_copy(win, hp.at[pl.ds(base, _PW), :])

            def row_body(r, acc):
                mxa = acc[0]
                sma = acc[1]
                den = acc[2]
                gi = base + r
                valid = (gi >= start) & (gi < start + n)
                mk = jnp.where(valid, m_v[pl.ds(gi, L)][0], 0.0)
                wm = (jnp.float32(1.0) if write_hp
                      else w_v[pl.ds(gi, L)][0])
                mxn = []
                smn_ = []
                for cth in range(nf):
                    v = win[r, pl.ds(cth * L, L)]
                    if not write_hp:
                        v = v * wm
                    mxn.append(jnp.where(mk != 0.0, jnp.maximum(mxa[cth], v),
                                         mxa[cth]))
                    smn_.append(sma[cth] + v * mk)
                return (tuple(mxn), tuple(smn_),
                        den + jnp.broadcast_to(mk, (L,)))

            return lax.fori_loop(0, _PW, row_body, accs)

        init = (tuple(neg for _ in range(nf)), tuple(zf for _ in range(nf)),
                zf)
        mxa, sma, den = lax.fori_loop(0, nwin, win_body, init)
        rden = 1.0 / jnp.maximum(den, 1.0)
        gl = g - core * 2 * NS
        for cth in range(nf):
            v = mxa[cth]
            win[0, pl.ds(cth * L, L)] = jnp.where(v > neg, v, zf)
            win[1, pl.ds(cth * L, L)] = sma[cth] * rden
        pltpu.sync_copy(win.at[pl.ds(0, 1), :], smx.at[pl.ds(gl, 1), :])
        pltpu.sync_copy(win.at[pl.ds(1, 1), :], smn.at[pl.ds(gl, 1), :])

    plsc.subcore_barrier()

    @pl.when(sid == 0)
    def _out():
        pltpu.sync_copy(smx, mx.at[pl.ds(core * 2 * NS, 2 * NS), :])
        pltpu.sync_copy(smn, mn.at[pl.ds(core * 2 * NS, 2 * NS), :])


def _sc_pool(table, wmul, mask, batch, write_hp):
    mesh = plsc.VectorSubcoreMesh(**_MESH)
    outs = [jax.ShapeDtypeStruct((B, FEAT), jnp.float32),
            jax.ShapeDtypeStruct((B, FEAT), jnp.float32)]
    if write_hp:
        outs.append(jax.ShapeDtypeStruct((N, FEAT), jnp.float32))

    if write_hp:
        def body(table_r, wmul_r, mask_r, batch_r, mx_r, mn_r, hp_r, b_v, w_v,
                 m_v, win, smx, smn):
            _pool_body(True, table_r, wmul_r, mask_r, batch_r, mx_r, mn_r,
                       hp_r, b_v, w_v, m_v, win, smx, smn)
    else:
        def body(table_r, wmul_r, mask_r, batch_r, mx_r, mn_r, b_v, w_v, m_v,
                 win, smx, smn):
            _pool_body(False, table_r, wmul_r, mask_r, batch_r, mx_r, mn_r,
                       None, b_v, w_v, m_v, win, smx, smn)

    fn = pl.kernel(
        body,
        out_type=tuple(outs),
        mesh=mesh,
        compiler_params=pltpu.CompilerParams(needs_layout_passes=False),
        scratch_types=[
            pltpu.VMEM((N + L,), jnp.int32),
            pltpu.VMEM((N + L,), jnp.float32),
            pltpu.VMEM((N + L,), jnp.float32),
            pltpu.VMEM((_PW, FEAT), jnp.float32),
            pltpu.VMEM_SHARED((2 * NS, FEAT), jnp.float32),
            pltpu.VMEM_SHARED((2 * NS, FEAT), jnp.float32),
        ],
    )
    return fn(table, wmul, mask, batch)


_KST = (N // NW) // 8 * 8
_KREM = N - NW * _KST
_GPT = B // NS


def _topk_body(keyv, batch, keep, kv_v, b_v, kbuf, skeep):
    core = lax.axis_index("c")
    sid = lax.axis_index("s")
    pltpu.sync_copy(keyv, kv_v.at[pl.ds(0, N)])
    pltpu.sync_copy(batch, b_v.at[pl.ds(0, N)])
    kv_v[pl.ds(N, L)] = jnp.full((L,), 3.4e38, jnp.float32)

    iota = lax.iota(jnp.int32, L)
    rots = [lax.rem(iota + r, L) for r in range(L)]

    for gg in range(0, _GPT, 2):
        g0 = sid * _GPT + gg
        g1 = g0 + 1
        start0, n0, lt1, n1 = _seg_bounds(b_v, g0, g1)

        for start, n in ((start0, n0), (lt1, n1)):
            k = (4 * n + 4) // 5
            nchunk = (n + L - 1) // L
            end = start + n

            def chunk_body(ci, _):
                p = jnp.maximum(start, jnp.minimum(start + ci * L, end - L))
                ids = iota + p
                kvv = kv_v[pl.ds(p, L)]

                def j_body(jc, cnt):
                    jbase = start + jc * L
                    jch = kv_v[pl.ds(jbase, L)]
                    for r in range(L):
                        kvj = _permute(jch, rots[r])
                        jjv = jbase + rots[r]
                        cond = ((kvj < kvv) | ((kvj == kvv) & (jjv < ids)))
                        cnt = cnt + cond.astype(jnp.int32)
                    return cnt

                cnt = lax.fori_loop(0, nchunk, j_body,
                                    jnp.zeros((L,), jnp.int32))
                keepv = (cnt < k).astype(jnp.float32)
                for r in range(L):
                    kbuf[r, pl.ds(0, L)] = jnp.broadcast_to(keepv[r], (L,))

                @pl.when(n >= L)
                def _full():
                    pltpu.sync_copy(kbuf, skeep.at[pl.ds(p, L), :])

                @pl.when(n < L)
                def _small():
                    for r in range(L):
                        @pl.when(r < n)
                        def _row():
                            pltpu.sync_copy(kbuf.at[pl.ds(r, 1), :],
                                            skeep.at[pl.ds(p + r, 1), :])

                return 0

            lax.fori_loop(0, nchunk, chunk_body, 0)

    plsc.subcore_barrier()

    wid = core * NS + sid
    obase = wid * _KST
    pltpu.sync_copy(skeep.at[pl.ds(obase, _KST), :],
                    keep.at[pl.ds(obase, _KST), :])

    @pl.when(wid == NW - 1)
    def _tail():
        tbase = NW * _KST
        pltpu.sync_copy(skeep.at[pl.ds(tbase, _KREM), :],
                        keep.at[pl.ds(tbase, _KREM), :])


def _sc_topk(keyv, batch):
    mesh = plsc.VectorSubcoreMesh(**_MESH)
    fn = pl.kernel(
        _topk_body,
        out_type=jax.ShapeDtypeStruct((N, L), jnp.float32),
        mesh=mesh,
        compiler_params=pltpu.CompilerParams(needs_layout_passes=False),
        scratch_types=[
            pltpu.VMEM((N + L,), jnp.float32),
            pltpu.VMEM((N + L,), jnp.int32),
            pltpu.VMEM((L, L), jnp.float32),
            pltpu.VMEM_SHARED((N + L, L), jnp.float32),
        ],
    )
    return fn(keyv, batch)


_DR = 1000


def _dense_tc_body(p_ref, x_ref, wrel_ref, wroot_ref, b_ref, keep_ref,
                   pw_ref, batch_ref, h_ref, score_ref, keyv_ref):
    agg = (p_ref[0] + p_ref[1]) * keep_ref[...]
    h = jnp.maximum(
        jnp.dot(agg, wrel_ref[...], preferred_element_type=jnp.float32)
        + b_ref[...]
        + jnp.dot(x_ref[...], wroot_ref[...],
                  preferred_element_type=jnp.float32),
        0.0)
    h_ref[...] = h
    pw = pw_ref[...]
    nrm = jnp.sqrt(jnp.sum(pw * pw))
    s = jnp.tanh(jnp.dot(h, pw, preferred_element_type=jnp.float32) / nrm)
    score_ref[...] = s
    keyv_ref[...] = batch_ref[...].astype(jnp.float32) * 4.0 - s


def _tc_dense(parts, x, w_rel, w_root, b, keep, pool_w, batch):
    grid = (N // _DR,)
    return pl.pallas_call(
        _dense_tc_body,
        grid=grid,
        in_specs=[
            pl.BlockSpec((NC, _DR, FEAT), lambda i: (0, i, 0)),
            pl.BlockSpec((_DR, FEAT), lambda i: (i, 0)),
            pl.BlockSpec((FEAT, FEAT), lambda i: (0, 0)),
            pl.BlockSpec((FEAT, FEAT), lambda i: (0, 0)),
            pl.BlockSpec((1, FEAT), lambda i: (0, 0)),
            pl.BlockSpec((_DR, 1), lambda i: (i, 0)),
            pl.BlockSpec((FEAT, 1), lambda i: (0, 0)),
            pl.BlockSpec((_DR, 1), lambda i: (i, 0)),
        ],
        out_specs=[
            pl.BlockSpec((_DR, FEAT), lambda i: (i, 0)),
            pl.BlockSpec((_DR, 1), lambda i: (i, 0)),
            pl.BlockSpec((_DR, 1), lambda i: (i, 0)),
        ],
        out_shape=[
            jax.ShapeDtypeStruct((N, FEAT), jnp.float32),
            jax.ShapeDtypeStruct((N, 1), jnp.float32),
            jax.ShapeDtypeStruct((N, 1), jnp.float32),
        ],
    )(parts, x, w_rel, w_root, b, keep, pool_w, batch)


def _mlp_body(mx1, mn1, mx2, mn2, mx3, mn3, l1w, l1b, l2w, l2b, l3w, l3b,
              out_ref):
    zmax = mx1[...] + mx2[...] + mx3[...]
    zmean = mn1[...] + mn2[...] + mn3[...]
    z = jnp.maximum(
        jnp.dot(zmax, l1w[:FEAT, :], preferred_element_type=jnp.float32)
        + jnp.dot(zmean, l1w[FEAT:, :], preferred_element_type=jnp.float32)
        + l1b[...], 0.0)
    z = jnp.maximum(
        jnp.dot(z, l2w[...], preferred_element_type=jnp.float32) + l2b[...],
        0.0)
    z = jnp.dot(z, l3w[...], preferred_element_type=jnp.float32) + l3b[...]
    m = jnp.max(z, axis=-1, keepdims=True)
    sh = z - m
    out_ref[...] = sh - jnp.log(jnp.sum(jnp.exp(sh), axis=-1, keepdims=True))


def _tc_mlp(mx1, mn1, mx2, mn2, mx3, mn3, l1w, l1b, l2w, l2b, l3w, l3b):
    return pl.pallas_call(
        _mlp_body,
        out_shape=jax.ShapeDtypeStruct((B, C), jnp.float32),
    )(mx1, mn1, mx2, mn2, mx3, mn3, l1w, l1b.reshape(1, -1), l2w,
      l2b.reshape(1, -1), l3w, l3b.reshape(1, -1))


def kernel(x, edge_index, batch, W1_rel, W1_root, b1, W2_rel, W2_root, b2,
           pool_w, W3_rel, W3_root, b3, L1W, L1b, L2W, L2b, L3W, L3b):
    src = edge_index[0]
    dst = edge_index[1]
    batch2 = batch.reshape(N, 1)
    ones1 = jnp.ones((N, 1), jnp.float32)
    ones = jnp.ones((N,), jnp.float32)
    pw2 = pool_w.reshape(FEAT, 1)

    p1 = _sc_conv(x, src, dst)
    h1, _, _ = _tc_dense(p1, x, W1_rel, W1_root, b1.reshape(1, FEAT), ones1,
                         pw2, batch2)
    mx1, mn1 = _sc_pool(h1, ones, ones, batch, False)

    p2 = _sc_conv(h1, src, dst)
    h2, score, keyv = _tc_dense(p2, h1, W2_rel, W2_root, b2.reshape(1, FEAT),
                                ones1, pw2, batch2)

    keep16 = _sc_topk(keyv.reshape(N), batch)
    keep = keep16[:, :1]
    keepf = keep16[:, 0]
    wmul2 = (score * keep).reshape(N)

    mx2, mn2, hp = _sc_pool(h2, wmul2, keepf, batch, True)

    p3 = _sc_conv(hp, src, dst)
    h3, _, _ = _tc_dense(p3, hp, W3_rel, W3_root, b3.reshape(1, FEAT), keep,
                         pw2, batch2)
    mx3, mn3 = _sc_pool(h3, keepf, keepf, batch, False)

    return _tc_mlp(mx1, mn1, mx2, mn2, mx3, mn3, L1W, L1b, L2W, L2b, L3W,
                   L3b)


# speedup vs baseline: 1.1779x; 1.1779x over previous
import functools

import jax
import jax.numpy as jnp
from jax import lax
from jax.experimental import pallas as pl
from jax.experimental.pallas import tpu as pltpu
from jax.experimental.pallas import tpu_sc as plsc

N = 10000
E = 320000
FEAT = 128
B = 64
C = 10

NC = 2
NS = 16
NW = NC * NS
L = 16

_MESH = dict(core_axis_name="c", subcore_axis_name="s", num_cores=NC,
             num_subcores=NS)


def _wid():
    return lax.axis_index("s") * NC + lax.axis_index("c")


def _permute(v, idx):
    dnums = lax.GatherDimensionNumbers(
        offset_dims=(), collapsed_slice_dims=(0,), start_index_map=(0,))
    return lax.gather(v, idx.reshape(L, 1), dnums, (1,),
                      mode=lax.GatherScatterMode.PROMISE_IN_BOUNDS)


_EPW = E // NW
_EC = 80
_NCHUNK = _EPW // _EC
_STRIPE = (N // NS) // 8 * 8
_REM = N - (NS - 1) * _STRIPE - _STRIPE


def _conv_kernel_body(table, src, dst, out, agg, srcv, dstv, rows0, rows1,
                      zb, sem0, sem1):
    core = lax.axis_index("c")
    sid = lax.axis_index("s")
    wid = sid * NC + core

    ebase = wid * _EPW
    pltpu.sync_copy(src.at[pl.ds(ebase, _EPW)], srcv)
    pltpu.sync_copy(dst.at[pl.ds(ebase, _EPW)], dstv)

    bufs = (rows0, rows1)
    sems = (sem0, sem1)

    def _start_g(q, b):
        pltpu.async_copy(table.at[srcv.at[pl.ds(q * _EC, _EC)]], bufs[b],
                         sems[b])

    def _wait_g(q, b):
        pltpu.make_async_copy(table.at[srcv.at[pl.ds(q * _EC, _EC)]],
                              bufs[b], sems[b]).wait()

    def _scatter(q, b):
        pltpu.sync_copy(bufs[b], agg.at[dstv.at[pl.ds(q * _EC, _EC)]],
                        add=True)

    _start_g(0, 0)

    for r in range(L):
        for cth in range(FEAT // L):
            zb[r, pl.ds(cth * L, L)] = jnp.zeros((L,), jnp.float32)

    zbase = sid * _STRIPE

    def _zero(i, _):
        pltpu.sync_copy(zb, agg.at[pl.ds(zbase + i * L, L), :])
        return 0

    nz = _STRIPE // L + jnp.where(sid == NS - 1, _REM // L, 0)
    lax.fori_loop(0, nz, _zero, 0)
    plsc.subcore_barrier()

    def _pair(i, _):
        q0 = 2 * i
        _start_g(q0 + 1, 1)
        _wait_g(q0, 0)
        _scatter(q0, 0)
        _start_g(q0 + 2, 0)
        _wait_g(q0 + 1, 1)
        _scatter(q0 + 1, 1)
        return 0

    lax.fori_loop(0, _NCHUNK // 2, _pair, 0)
    _wait_g(_NCHUNK - 1, 0)
    _scatter(_NCHUNK - 1, 0)
    plsc.subcore_barrier()

    obase = sid * _STRIPE
    pltpu.sync_copy(agg.at[pl.ds(obase, _STRIPE), :],
                    out.at[core, pl.ds(obase, _STRIPE), :])

    @pl.when(sid == NS - 1)
    def _tail():
        tbase = NS * _STRIPE
        pltpu.sync_copy(agg.at[pl.ds(tbase, _REM), :],
                        out.at[core, pl.ds(tbase, _REM), :])


def _sc_conv(table, src, dst):
    mesh = plsc.VectorSubcoreMesh(**_MESH)

    fn = pl.kernel(
        _conv_kernel_body,
        out_type=jax.ShapeDtypeStruct((NC, N, FEAT), jnp.float32),
        mesh=mesh,
        compiler_params=pltpu.CompilerParams(needs_layout_passes=False),
        scratch_types=[
            pltpu.VMEM_SHARED((N, FEAT), jnp.float32),
            pltpu.VMEM((_EPW,), jnp.int32),
            pltpu.VMEM((_EPW,), jnp.int32),
            pltpu.VMEM((_EC, FEAT), jnp.float32),
            pltpu.VMEM((_EC, FEAT), jnp.float32),
            pltpu.VMEM((L, FEAT), jnp.float32),
            pltpu.SemaphoreType.DMA,
            pltpu.SemaphoreType.DMA,
        ],
    )
    return fn(table, src, dst)


def _seg_bounds(b_v, g0, g1):
    zero = jnp.zeros((L,), jnp.int32)

    def body(i, carry):
        lt0, eq0, lt1, eq1 = carry
        ch = b_v[pl.ds(i * L, L)]
        lt0 = lt0 + plsc.all_reduce_population_count(ch < g0)
        eq0 = eq0 + plsc.all_reduce_population_count(ch == g0)
        lt1 = lt1 + plsc.all_reduce_population_count(ch < g1)
        eq1 = eq1 + plsc.all_reduce_population_count(ch == g1)
        return lt0, eq0, lt1, eq1

    lt0, eq0, lt1, eq1 = lax.fori_loop(0, N // L, body,
                                       (zero, zero, zero, zero))
    return (jnp.max(lt0), jnp.max(eq0), jnp.max(lt1), jnp.max(eq1))


_PW = 128


def _pool_body(write_hp, table, wmul, mask, batch, mx, mn, hp, b_v, w_v, m_v,
               win, smx, smn):
    core = lax.axis_index("c")
    sid = lax.axis_index("s")
    wid = core * NS + sid
    g0 = wid * 2
    g1 = g0 + 1
    pltpu.sync_copy(batch, b_v.at[pl.ds(0, N)])
    pltpu.sync_copy(wmul, w_v.at[pl.ds(0, N)])
    pltpu.sync_copy(mask, m_v.at[pl.ds(0, N)])
    start0, n0, lt1, n1 = _seg_bounds(b_v, g0, g1)
    start1 = lt1

    nf = FEAT // L
    neg = jnp.full((L,), -jnp.inf, jnp.float32)
    zf = jnp.zeros((L,), jnp.float32)

    for g, start, n in ((g0, start0, n0), (g1, start1, n1)):
        astart = start // 8 * 8
        nwin = (start - astart + n + _PW - 1) // _PW

        def win_body(w, carry):
            accs = carry
            a = astart + w * _PW
            base = jnp.minimum(a, N - _PW)
            pltpu.sync_copy(table.at[pl.ds(base, _PW), :], win)

            if write_hp:
                def hp_row(r, _):
                    wm = w_v[pl.ds(base + r, L)][0]
                    for cth in range(nf):
                        v = win[r, pl.ds(cth * L, L)]
                        win[r, pl.ds(cth * L, L)] = v * wm
                    return 0

                lax.fori_loop(0, _PW, hp_row, 0)
                pltpu.sync_copy(win, hp.at[pl.ds(base, _PW), :])

            def row_body(r, acc):
                mxa = acc[0]
                sma = acc[1]
                den = acc[2]
                gi = base + r
                valid = (gi >= start) & (gi < start + n)
                mk = jnp.where(valid, m_v[pl.ds(gi, L)][0], 0.0)
                wm = (jnp.float32(1.0) if write_hp
                      else w_v[pl.ds(gi, L)][0])
                mxn = []
                smn_ = []
                for cth in range(nf):
                    v = win[r, pl.ds(cth * L, L)]
                    if not write_hp:
                        v = v * wm
                    mxn.append(jnp.where(mk != 0.0, jnp.maximum(mxa[cth], v),
                                         mxa[cth]))
                    smn_.append(sma[cth] + v * mk)
                return (tuple(mxn), tuple(smn_),
                        den + jnp.broadcast_to(mk, (L,)))

            return lax.fori_loop(0, _PW, row_body, accs)

        init = (tuple(neg for _ in range(nf)), tuple(zf for _ in range(nf)),
                zf)
        mxa, sma, den = lax.fori_loop(0, nwin, win_body, init)
        rden = 1.0 / jnp.maximum(den, 1.0)
        gl = g - core * 2 * NS
        for cth in range(nf):
            v = mxa[cth]
            win[0, pl.ds(cth * L, L)] = jnp.where(v > neg, v, zf)
            win[1, pl.ds(cth * L, L)] = sma[cth] * rden
        pltpu.sync_copy(win.at[pl.ds(0, 1), :], smx.at[pl.ds(gl, 1), :])
        pltpu.sync_copy(win.at[pl.ds(1, 1), :], smn.at[pl.ds(gl, 1), :])

    plsc.subcore_barrier()

    @pl.when(sid == 0)
    def _out():
        pltpu.sync_copy(smx, mx.at[pl.ds(core * 2 * NS, 2 * NS), :])
        pltpu.sync_copy(smn, mn.at[pl.ds(core * 2 * NS, 2 * NS), :])


def _sc_pool(table, wmul, mask, batch, write_hp):
    mesh = plsc.VectorSubcoreMesh(**_MESH)
    outs = [jax.ShapeDtypeStruct((B, FEAT), jnp.float32),
            jax.ShapeDtypeStruct((B, FEAT), jnp.float32)]
    if write_hp:
        outs.append(jax.ShapeDtypeStruct((N, FEAT), jnp.float32))

    if write_hp:
        def body(table_r, wmul_r, mask_r, batch_r, mx_r, mn_r, hp_r, b_v, w_v,
                 m_v, win, smx, smn):
            _pool_body(True, table_r, wmul_r, mask_r, batch_r, mx_r, mn_r,
                       hp_r, b_v, w_v, m_v, win, smx, smn)
    else:
        def body(table_r, wmul_r, mask_r, batch_r, mx_r, mn_r, b_v, w_v, m_v,
                 win, smx, smn):
            _pool_body(False, table_r, wmul_r, mask_r, batch_r, mx_r, mn_r,
                       None, b_v, w_v, m_v, win, smx, smn)

    fn = pl.kernel(
        body,
        out_type=tuple(outs),
        mesh=mesh,
        compiler_params=pltpu.CompilerParams(needs_layout_passes=False),
        scratch_types=[
            pltpu.VMEM((N + L,), jnp.int32),
            pltpu.VMEM((N + L,), jnp.float32),
            pltpu.VMEM((N + L,), jnp.float32),
            pltpu.VMEM((_PW, FEAT), jnp.float32),
            pltpu.VMEM_SHARED((2 * NS, FEAT), jnp.float32),
            pltpu.VMEM_SHARED((2 * NS, FEAT), jnp.float32),
        ],
    )
    return fn(table, wmul, mask, batch)


_KST = (N // NW) // 8 * 8
_KREM = N - NW * _KST
_GPT = B // NS


def _topk_body(keyv, batch, keep, kv_v, b_v, kbuf, skeep):
    core = lax.axis_index("c")
    sid = lax.axis_index("s")
    pltpu.sync_copy(keyv, kv_v.at[pl.ds(0, N)])
    pltpu.sync_copy(batch, b_v.at[pl.ds(0, N)])
    kv_v[pl.ds(N, L)] = jnp.full((L,), 3.4e38, jnp.float32)

    iota = lax.iota(jnp.int32, L)
    rots = [lax.rem(iota + r, L) for r in range(L)]

    for gg in range(0, _GPT, 2):
        g0 = sid * _GPT + gg
        g1 = g0 + 1
        start0, n0, lt1, n1 = _seg_bounds(b_v, g0, g1)

        for start, n in ((start0, n0), (lt1, n1)):
            k = (4 * n + 4) // 5
            nchunk = (n + L - 1) // L
            end = start + n

            def chunk_body(ci, _):
                p = jnp.maximum(start, jnp.minimum(start + ci * L, end - L))
                ids = iota + p
                kvv = kv_v[pl.ds(p, L)]

                def j_body(jc, cnt):
                    jbase = start + jc * L
                    jch = kv_v[pl.ds(jbase, L)]
                    for r in range(L):
                        kvj = _permute(jch, rots[r])
                        jjv = jbase + rots[r]
                        cond = ((kvj < kvv) | ((kvj == kvv) & (jjv < ids)))
                        cnt = cnt + cond.astype(jnp.int32)
                    return cnt

                cnt = lax.fori_loop(0, nchunk, j_body,
                                    jnp.zeros((L,), jnp.int32))
                keepv = (cnt < k).astype(jnp.float32)
                for r in range(L):
                    kbuf[r, pl.ds(0, L)] = jnp.broadcast_to(keepv[r], (L,))

                @pl.when(n >= L)
                def _full():
                    pltpu.sync_copy(kbuf, skeep.at[pl.ds(p, L), :])

                @pl.when(n < L)
                def _small():
                    for r in range(L):
                        @pl.when(r < n)
                        def _row():
                            pltpu.sync_copy(kbuf.at[pl.ds(r, 1), :],
                                            skeep.at[pl.ds(p + r, 1), :])

                return 0

            lax.fori_loop(0, nchunk, chunk_body, 0)

    plsc.subcore_barrier()

    wid = core * NS + sid
    obase = wid * _KST
    pltpu.sync_copy(skeep.at[pl.ds(obase, _KST), :],
                    keep.at[pl.ds(obase, _KST), :])

    @pl.when(wid == NW - 1)
    def _tail():
        tbase = NW * _KST
        pltpu.sync_copy(skeep.at[pl.ds(tbase, _KREM), :],
                        keep.at[pl.ds(tbase, _KREM), :])


def _sc_topk(keyv, batch):
    mesh = plsc.VectorSubcoreMesh(**_MESH)
    fn = pl.kernel(
        _topk_body,
        out_type=jax.ShapeDtypeStruct((N, L), jnp.float32),
        mesh=mesh,
        compiler_params=pltpu.CompilerParams(needs_layout_passes=False),
        scratch_types=[
            pltpu.VMEM((N + L,), jnp.float32),
            pltpu.VMEM((N + L,), jnp.int32),
            pltpu.VMEM((L, L), jnp.float32),
            pltpu.VMEM_SHARED((N + L, L), jnp.float32),
        ],
    )
    return fn(keyv, batch)


_DR = 1000


def _dense_tc_body(p_ref, x_ref, wrel_ref, wroot_ref, b_ref, keep_ref,
                   pw_ref, batch_ref, h_ref, score_ref, keyv_ref):
    agg = (p_ref[0] + p_ref[1]) * keep_ref[...]
    h = jnp.maximum(
        jnp.dot(agg, wrel_ref[...], preferred_element_type=jnp.float32)
        + b_ref[...]
        + jnp.dot(x_ref[...], wroot_ref[...],
                  preferred_element_type=jnp.float32),
        0.0)
    h_ref[...] = h
    pw = pw_ref[...]
    nrm = jnp.sqrt(jnp.sum(pw * pw))
    s = jnp.tanh(jnp.dot(h, pw, preferred_element_type=jnp.float32) / nrm)
    score_ref[...] = s
    keyv_ref[...] = batch_ref[...].astype(jnp.float32) * 4.0 - s


def _tc_dense(parts, x, w_rel, w_root, b, keep, pool_w, batch):
    grid = (N // _DR,)
    return pl.pallas_call(
        _dense_tc_body,
        grid=grid,
        in_specs=[
            pl.BlockSpec((NC, _DR, FEAT), lambda i: (0, i, 0)),
            pl.BlockSpec((_DR, FEAT), lambda i: (i, 0)),
            pl.BlockSpec((FEAT, FEAT), lambda i: (0, 0)),
            pl.BlockSpec((FEAT, FEAT), lambda i: (0, 0)),
            pl.BlockSpec((1, FEAT), lambda i: (0, 0)),
            pl.BlockSpec((_DR, 1), lambda i: (i, 0)),
            pl.BlockSpec((FEAT, 1), lambda i: (0, 0)),
            pl.BlockSpec((_DR, 1), lambda i: (i, 0)),
        ],
        out_specs=[
            pl.BlockSpec((_DR, FEAT), lambda i: (i, 0)),
            pl.BlockSpec((_DR, 1), lambda i: (i, 0)),
            pl.BlockSpec((_DR, 1), lambda i: (i, 0)),
        ],
        out_shape=[
            jax.ShapeDtypeStruct((N, FEAT), jnp.float32),
            jax.ShapeDtypeStruct((N, 1), jnp.float32),
            jax.ShapeDtypeStruct((N, 1), jnp.float32),
        ],
    )(parts, x, w_rel, w_root, b, keep, pool_w, batch)


def _mlp_body(mx1, mn1, mx2, mn2, mx3, mn3, l1w, l1b, l2w, l2b, l3w, l3b,
              out_ref):
    zmax = mx1[...] + mx2[...] + mx3[...]
    zmean = mn1[...] + mn2[...] + mn3[...]
    z = jnp.maximum(
        jnp.dot(zmax, l1w[:FEAT, :], preferred_element_type=jnp.float32)
        + jnp.dot(zmean, l1w[FEAT:, :], preferred_element_type=jnp.float32)
        + l1b[...], 0.0)
    z = jnp.maximum(
        jnp.dot(z, l2w[...], preferred_element_type=jnp.float32) + l2b[...],
        0.0)
    z = jnp.dot(z, l3w[...], preferred_element_type=jnp.float32) + l3b[...]
    m = jnp.max(z, axis=-1, keepdims=True)
    sh = z - m
    out_ref[...] = sh - jnp.log(jnp.sum(jnp.exp(sh), axis=-1, keepdims=True))


def _tc_mlp(mx1, mn1, mx2, mn2, mx3, mn3, l1w, l1b, l2w, l2b, l3w, l3b):
    return pl.pallas_call(
        _mlp_body,
        out_shape=jax.ShapeDtypeStruct((B, C), jnp.float32),
    )(mx1, mn1, mx2, mn2, mx3, mn3, l1w, l1b.reshape(1, -1), l2w,
      l2b.reshape(1, -1), l3w, l3b.reshape(1, -1))


def kernel(x, edge_index, batch, W1_rel, W1_root, b1, W2_rel, W2_root, b2,
           pool_w, W3_rel, W3_root, b3, L1W, L1b, L2W, L2b, L3W, L3b):
    src = edge_index[0]
    dst = edge_index[1]
    batch2 = batch.reshape(N, 1)
    ones1 = jnp.ones((N, 1), jnp.float32)
    ones = jnp.ones((N,), jnp.float32)
    pw2 = pool_w.reshape(FEAT, 1)

    p1 = _sc_conv(x, src, dst)
    h1, _, _ = _tc_dense(p1, x, W1_rel, W1_root, b1.reshape(1, FEAT), ones1,
                         pw2, batch2)
    mx1, mn1 = _sc_pool(h1, ones, ones, batch, False)

    p2 = _sc_conv(h1, src, dst)
    h2, score, keyv = _tc_dense(p2, h1, W2_rel, W2_root, b2.reshape(1, FEAT),
                                ones1, pw2, batch2)

    keep16 = _sc_topk(keyv.reshape(N), batch)
    keep = keep16[:, :1]
    keepf = keep16[:, 0]
    wmul2 = (score * keep).reshape(N)

    mx2, mn2, hp = _sc_pool(h2, wmul2, keepf, batch, True)

    p3 = _sc_conv(hp, src, dst)
    h3, _, _ = _tc_dense(p3, hp, W3_rel, W3_root, b3.reshape(1, FEAT), keep,
                         pw2, batch2)
    mx3, mn3 = _sc_pool(h3, keepf, keepf, batch, False)

    return _tc_mlp(mx1, mn1, mx2, mn2, mx3, mn3, L1W, L1b, L2W, L2b, L3W,
                   L3b)


# speedup vs baseline: 1.2217x; 1.0371x over previous
import functools

import jax
import jax.numpy as jnp
from jax import lax
from jax.experimental import pallas as pl
from jax.experimental.pallas import tpu as pltpu
from jax.experimental.pallas import tpu_sc as plsc

N = 10000
E = 320000
FEAT = 128
B = 64
C = 10

NC = 2
NS = 16
NW = NC * NS
L = 16

_MESH = dict(core_axis_name="c", subcore_axis_name="s", num_cores=NC,
             num_subcores=NS)


def _wid():
    return lax.axis_index("s") * NC + lax.axis_index("c")


def _permute(v, idx):
    dnums = lax.GatherDimensionNumbers(
        offset_dims=(), collapsed_slice_dims=(0,), start_index_map=(0,))
    return lax.gather(v, idx.reshape(L, 1), dnums, (1,),
                      mode=lax.GatherScatterMode.PROMISE_IN_BOUNDS)


_EPW = E // NW
_EC = 96
_NCHUNK = _EPW // _EC
_ETAIL = _EPW - _NCHUNK * _EC
_STRIPE = (N // NS) // 8 * 8
_REM = N - (NS - 1) * _STRIPE - _STRIPE


def _conv_kernel_body(table, src, dst, out, agg, srcv, dstv, rows0, rows1,
                      zb, sem0, sem1):
    core = lax.axis_index("c")
    sid = lax.axis_index("s")
    wid = sid * NC + core

    ebase = wid * _EPW
    pltpu.sync_copy(src.at[pl.ds(ebase, _EPW)], srcv)
    pltpu.sync_copy(dst.at[pl.ds(ebase, _EPW)], dstv)

    bufs = (rows0, rows1)
    sems = (sem0, sem1)

    def _start_g(q, b):
        pltpu.async_copy(table.at[srcv.at[pl.ds(q * _EC, _EC)]], bufs[b],
                         sems[b])

    def _wait_g(q, b):
        pltpu.make_async_copy(table.at[srcv.at[pl.ds(q * _EC, _EC)]],
                              bufs[b], sems[b]).wait()

    def _scatter(q, b):
        pltpu.sync_copy(bufs[b], agg.at[dstv.at[pl.ds(q * _EC, _EC)]],
                        add=True)

    _start_g(0, 0)

    for r in range(L):
        for cth in range(FEAT // L):
            zb[r, pl.ds(cth * L, L)] = jnp.zeros((L,), jnp.float32)

    zbase = sid * _STRIPE

    def _zero(i, _):
        pltpu.sync_copy(zb, agg.at[pl.ds(zbase + i * L, L), :])
        return 0

    nz = _STRIPE // L + jnp.where(sid == NS - 1, _REM // L, 0)
    lax.fori_loop(0, nz, _zero, 0)
    plsc.subcore_barrier()

    def _pair(i, _):
        q0 = 2 * i
        _start_g(q0 + 1, 1)
        _wait_g(q0, 0)
        _scatter(q0, 0)

        @pl.when(q0 + 2 < _NCHUNK)
        def _g():
            _start_g(q0 + 2, 0)

        _wait_g(q0 + 1, 1)
        _scatter(q0 + 1, 1)
        return 0

    lax.fori_loop(0, _NCHUNK // 2, _pair, 0)

    tb = _NCHUNK * _EC
    pltpu.async_copy(table.at[srcv.at[pl.ds(tb, _ETAIL)]],
                     rows0.at[pl.ds(0, _ETAIL), :], sem0)
    pltpu.make_async_copy(table.at[srcv.at[pl.ds(tb, _ETAIL)]],
                          rows0.at[pl.ds(0, _ETAIL), :], sem0).wait()
    pltpu.sync_copy(rows0.at[pl.ds(0, _ETAIL), :],
                    agg.at[dstv.at[pl.ds(tb, _ETAIL)]], add=True)
    plsc.subcore_barrier()

    obase = sid * _STRIPE
    pltpu.sync_copy(agg.at[pl.ds(obase, _STRIPE), :],
                    out.at[core, pl.ds(obase, _STRIPE), :])

    @pl.when(sid == NS - 1)
    def _tail():
        tbase = NS * _STRIPE
        pltpu.sync_copy(agg.at[pl.ds(tbase, _REM), :],
                        out.at[core, pl.ds(tbase, _REM), :])


def _sc_conv(table, src, dst):
    mesh = plsc.VectorSubcoreMesh(**_MESH)

    fn = pl.kernel(
        _conv_kernel_body,
        out_type=jax.ShapeDtypeStruct((NC, N, FEAT), jnp.float32),
        mesh=mesh,
        compiler_params=pltpu.CompilerParams(needs_layout_passes=False),
        scratch_types=[
            pltpu.VMEM_SHARED((N, FEAT), jnp.float32),
            pltpu.VMEM((_EPW,), jnp.int32),
            pltpu.VMEM((_EPW,), jnp.int32),
            pltpu.VMEM((_EC, FEAT), jnp.float32),
            pltpu.VMEM((_EC, FEAT), jnp.float32),
            pltpu.VMEM((L, FEAT), jnp.float32),
            pltpu.SemaphoreType.DMA,
            pltpu.SemaphoreType.DMA,
        ],
    )
    return fn(table, src, dst)


def _lb(b_v, g):
    def body(_, carry):
        lo, hi = carry
        mid = (lo + hi) // 2
        v = b_v[pl.ds(mid, L)][0]
        pred = v < g
        return (jnp.where(pred, mid + 1, lo), jnp.where(pred, hi, mid))

    lo, _ = lax.fori_loop(0, 14, body, (jnp.int32(0), jnp.int32(N)))
    return lo


_PW = 128


def _pool_body(write_hp, table, wmul, mask, batch, mx, mn, hp, b_v, w_v, m_v,
               win, smx, smn):
    core = lax.axis_index("c")
    sid = lax.axis_index("s")
    wid = core * NS + sid
    g0 = wid * 2
    g1 = g0 + 1
    pltpu.sync_copy(batch, b_v.at[pl.ds(0, N)])
    pltpu.sync_copy(wmul, w_v.at[pl.ds(0, N)])
    pltpu.sync_copy(mask, m_v.at[pl.ds(0, N)])
    start0 = _lb(b_v, g0)
    start1 = _lb(b_v, g1)
    end1 = _lb(b_v, g1 + 1)
    n0 = start1 - start0
    n1 = end1 - start1

    nf = FEAT // L
    neg = jnp.full((L,), -jnp.inf, jnp.float32)
    zf = jnp.zeros((L,), jnp.float32)

    for g, start, n in ((g0, start0, n0), (g1, start1, n1)):
        astart = start // 8 * 8
        nwin = (start - astart + n + _PW - 1) // _PW

        def win_body(w, carry):
            accs = carry
            a = astart + w * _PW
            base = jnp.minimum(a, N - _PW)
            pltpu.sync_copy(table.at[pl.ds(base, _PW), :], win)

            if write_hp:
                def hp_row(r, _):
                    wm = w_v[pl.ds(base + r, L)][0]
                    for cth in range(nf):
                        v = win[r, pl.ds(cth * L, L)]
                        win[r, pl.ds(cth * L, L)] = v * wm
                    return 0

                lax.fori_loop(0, _PW, hp_row, 0)
                pltpu.sync_copy(win, hp.at[pl.ds(base, _PW), :])

            def row_body(r, acc):
                mxa = acc[0]
                sma = acc[1]
                den = acc[2]
                gi = base + r
                valid = (gi >= start) & (gi < start + n)
                mk = jnp.where(valid, m_v[pl.ds(gi, L)][0], 0.0)
                wm = (jnp.float32(1.0) if write_hp
                      else w_v[pl.ds(gi, L)][0])
                mxn = []
                smn_ = []
                for cth in range(nf):
                    v = win[r, pl.ds(cth * L, L)]
                    if not write_hp:
                        v = v * wm
                    mxn.append(jnp.where(mk != 0.0, jnp.maximum(mxa[cth], v),
                                         mxa[cth]))
                    smn_.append(sma[cth] + v * mk)
                return (tuple(mxn), tuple(smn_),
                        den + jnp.broadcast_to(mk, (L,)))

            return lax.fori_loop(0, _PW, row_body, accs)

        init = (tuple(neg for _ in range(nf)), tuple(zf for _ in range(nf)),
                zf)
        mxa, sma, den = lax.fori_loop(0, nwin, win_body, init)
        rden = 1.0 / jnp.maximum(den, 1.0)
        gl = g - core * 2 * NS
        for cth in range(nf):
            v = mxa[cth]
            win[0, pl.ds(cth * L, L)] = jnp.where(v > neg, v, zf)
            win[1, pl.ds(cth * L, L)] = sma[cth] * rden
        pltpu.sync_copy(win.at[pl.ds(0, 1), :], smx.at[pl.ds(gl, 1), :])
        pltpu.sync_copy(win.at[pl.ds(1, 1), :], smn.at[pl.ds(gl, 1), :])

    plsc.subcore_barrier()

    @pl.when(sid == 0)
    def _out():
        pltpu.sync_copy(smx, mx.at[pl.ds(core * 2 * NS, 2 * NS), :])
        pltpu.sync_copy(smn, mn.at[pl.ds(core * 2 * NS, 2 * NS), :])


def _sc_pool(table, wmul, mask, batch, write_hp):
    mesh = plsc.VectorSubcoreMesh(**_MESH)
    outs = [jax.ShapeDtypeStruct((B, FEAT), jnp.float32),
            jax.ShapeDtypeStruct((B, FEAT), jnp.float32)]
    if write_hp:
        outs.append(jax.ShapeDtypeStruct((N, FEAT), jnp.float32))

    if write_hp:
        def body(table_r, wmul_r, mask_r, batch_r, mx_r, mn_r, hp_r, b_v, w_v,
                 m_v, win, smx, smn):
            _pool_body(True, table_r, wmul_r, mask_r, batch_r, mx_r, mn_r,
                       hp_r, b_v, w_v, m_v, win, smx, smn)
    else:
        def body(table_r, wmul_r, mask_r, batch_r, mx_r, mn_r, b_v, w_v, m_v,
                 win, smx, smn):
            _pool_body(False, table_r, wmul_r, mask_r, batch_r, mx_r, mn_r,
                       None, b_v, w_v, m_v, win, smx, smn)

    fn = pl.kernel(
        body,
        out_type=tuple(outs),
        mesh=mesh,
        compiler_params=pltpu.CompilerParams(needs_layout_passes=False),
        scratch_types=[
            pltpu.VMEM((N + L,), jnp.int32),
            pltpu.VMEM((N + L,), jnp.float32),
            pltpu.VMEM((N + L,), jnp.float32),
            pltpu.VMEM((_PW, FEAT), jnp.float32),
            pltpu.VMEM_SHARED((2 * NS, FEAT), jnp.float32),
            pltpu.VMEM_SHARED((2 * NS, FEAT), jnp.float32),
        ],
    )
    return fn(table, wmul, mask, batch)


_KST = (N // NW) // 8 * 8
_KREM = N - NW * _KST
_GPT = B // NS


def _topk_body(keyv, batch, keep, kv_v, b_v, kbuf, skeep):
    core = lax.axis_index("c")
    sid = lax.axis_index("s")
    pltpu.sync_copy(keyv, kv_v.at[pl.ds(0, N)])
    pltpu.sync_copy(batch, b_v.at[pl.ds(0, N)])
    kv_v[pl.ds(N, L)] = jnp.full((L,), 3.4e38, jnp.float32)

    iota = lax.iota(jnp.int32, L)
    rots = [lax.rem(iota + r, L) for r in range(L)]

    gbase = sid * _GPT
    bounds = [_lb(b_v, gbase + i) for i in range(_GPT + 1)]

    for gg in range(_GPT):
        start = bounds[gg]
        n = bounds[gg + 1] - start

        if True:
            k = (4 * n + 4) // 5
            nchunk = (n + L - 1) // L
            end = start + n

            def chunk_body(ci, _):
                p = jnp.maximum(start, jnp.minimum(start + ci * L, end - L))
                ids = iota + p
                kvv = kv_v[pl.ds(p, L)]

                def j_body(jc, cnt):
                    jbase = start + jc * L
                    jch = kv_v[pl.ds(jbase, L)]
                    for r in range(L):
                        kvj = _permute(jch, rots[r])
                        jjv = jbase + rots[r]
                        cond = ((kvj < kvv) | ((kvj == kvv) & (jjv < ids)))
                        cnt = cnt + cond.astype(jnp.int32)
                    return cnt

                cnt = lax.fori_loop(0, nchunk, j_body,
                                    jnp.zeros((L,), jnp.int32))
                keepv = (cnt < k).astype(jnp.float32)
                for r in range(L):
                    kbuf[r, pl.ds(0, L)] = jnp.broadcast_to(keepv[r], (L,))

                @pl.when(n >= L)
                def _full():
                    pltpu.sync_copy(kbuf, skeep.at[pl.ds(p, L), :])

                @pl.when(n < L)
                def _small():
                    for r in range(L):
                        @pl.when(r < n)
                        def _row():
                            pltpu.sync_copy(kbuf.at[pl.ds(r, 1), :],
                                            skeep.at[pl.ds(p + r, 1), :])

                return 0

            lax.fori_loop(0, nchunk, chunk_body, 0)

    plsc.subcore_barrier()

    wid = core * NS + sid
    obase = wid * _KST
    pltpu.sync_copy(skeep.at[pl.ds(obase, _KST), :],
                    keep.at[pl.ds(obase, _KST), :])

    @pl.when(wid == NW - 1)
    def _tail():
        tbase = NW * _KST
        pltpu.sync_copy(skeep.at[pl.ds(tbase, _KREM), :],
                        keep.at[pl.ds(tbase, _KREM), :])


def _sc_topk(keyv, batch):
    mesh = plsc.VectorSubcoreMesh(**_MESH)
    fn = pl.kernel(
        _topk_body,
        out_type=jax.ShapeDtypeStruct((N, L), jnp.float32),
        mesh=mesh,
        compiler_params=pltpu.CompilerParams(needs_layout_passes=False),
        scratch_types=[
            pltpu.VMEM((N + L,), jnp.float32),
            pltpu.VMEM((N + L,), jnp.int32),
            pltpu.VMEM((L, L), jnp.float32),
            pltpu.VMEM_SHARED((N + L, L), jnp.float32),
        ],
    )
    return fn(keyv, batch)


_DR = 1000


def _dense_tc_body(p_ref, x_ref, wrel_ref, wroot_ref, b_ref, keep_ref,
                   pw_ref, batch_ref, h_ref, score_ref, keyv_ref):
    agg = (p_ref[0] + p_ref[1]) * keep_ref[...]
    h = jnp.maximum(
        jnp.dot(agg, wrel_ref[...], preferred_element_type=jnp.float32)
        + b_ref[...]
        + jnp.dot(x_ref[...], wroot_ref[...],
                  preferred_element_type=jnp.float32),
        0.0)
    h_ref[...] = h
    pw = pw_ref[...]
    nrm = jnp.sqrt(jnp.sum(pw * pw))
    s = jnp.tanh(jnp.dot(h, pw, preferred_element_type=jnp.float32) / nrm)
    score_ref[...] = s
    keyv_ref[...] = batch_ref[...].astype(jnp.float32) * 4.0 - s


def _tc_dense(parts, x, w_rel, w_root, b, keep, pool_w, batch):
    grid = (N // _DR,)
    return pl.pallas_call(
        _dense_tc_body,
        grid=grid,
        in_specs=[
            pl.BlockSpec((NC, _DR, FEAT), lambda i: (0, i, 0)),
            pl.BlockSpec((_DR, FEAT), lambda i: (i, 0)),
            pl.BlockSpec((FEAT, FEAT), lambda i: (0, 0)),
            pl.BlockSpec((FEAT, FEAT), lambda i: (0, 0)),
            pl.BlockSpec((1, FEAT), lambda i: (0, 0)),
            pl.BlockSpec((_DR, 1), lambda i: (i, 0)),
            pl.BlockSpec((FEAT, 1), lambda i: (0, 0)),
            pl.BlockSpec((_DR, 1), lambda i: (i, 0)),
        ],
        out_specs=[
            pl.BlockSpec((_DR, FEAT), lambda i: (i, 0)),
            pl.BlockSpec((_DR, 1), lambda i: (i, 0)),
            pl.BlockSpec((_DR, 1), lambda i: (i, 0)),
        ],
        out_shape=[
            jax.ShapeDtypeStruct((N, FEAT), jnp.float32),
            jax.ShapeDtypeStruct((N, 1), jnp.float32),
            jax.ShapeDtypeStruct((N, 1), jnp.float32),
        ],
    )(parts, x, w_rel, w_root, b, keep, pool_w, batch)


def _mlp_body(mx1, mn1, mx2, mn2, mx3, mn3, l1w, l1b, l2w, l2b, l3w, l3b,
              out_ref):
    zmax = mx1[...] + mx2[...] + mx3[...]
    zmean = mn1[...] + mn2[...] + mn3[...]
    z = jnp.maximum(
        jnp.dot(zmax, l1w[:FEAT, :], preferred_element_type=jnp.float32)
        + jnp.dot(zmean, l1w[FEAT:, :], preferred_element_type=jnp.float32)
        + l1b[...], 0.0)
    z = jnp.maximum(
        jnp.dot(z, l2w[...], preferred_element_type=jnp.float32) + l2b[...],
        0.0)
    z = jnp.dot(z, l3w[...], preferred_element_type=jnp.float32) + l3b[...]
    m = jnp.max(z, axis=-1, keepdims=True)
    sh = z - m
    out_ref[...] = sh - jnp.log(jnp.sum(jnp.exp(sh), axis=-1, keepdims=True))


def _tc_mlp(mx1, mn1, mx2, mn2, mx3, mn3, l1w, l1b, l2w, l2b, l3w, l3b):
    return pl.pallas_call(
        _mlp_body,
        out_shape=jax.ShapeDtypeStruct((B, C), jnp.float32),
    )(mx1, mn1, mx2, mn2, mx3, mn3, l1w, l1b.reshape(1, -1), l2w,
      l2b.reshape(1, -1), l3w, l3b.reshape(1, -1))


def kernel(x, edge_index, batch, W1_rel, W1_root, b1, W2_rel, W2_root, b2,
           pool_w, W3_rel, W3_root, b3, L1W, L1b, L2W, L2b, L3W, L3b):
    src = edge_index[0]
    dst = edge_index[1]
    batch2 = batch.reshape(N, 1)
    ones1 = jnp.ones((N, 1), jnp.float32)
    ones = jnp.ones((N,), jnp.float32)
    pw2 = pool_w.reshape(FEAT, 1)

    p1 = _sc_conv(x, src, dst)
    h1, _, _ = _tc_dense(p1, x, W1_rel, W1_root, b1.reshape(1, FEAT), ones1,
                         pw2, batch2)
    mx1, mn1 = _sc_pool(h1, ones, ones, batch, False)

    p2 = _sc_conv(h1, src, dst)
    h2, score, keyv = _tc_dense(p2, h1, W2_rel, W2_root, b2.reshape(1, FEAT),
                                ones1, pw2, batch2)

    keep16 = _sc_topk(keyv.reshape(N), batch)
    keep = keep16[:, :1]
    keepf = keep16[:, 0]
    wmul2 = (score * keep).reshape(N)

    mx2, mn2, hp = _sc_pool(h2, wmul2, keepf, batch, True)

    p3 = _sc_conv(hp, src, dst)
    h3, _, _ = _tc_dense(p3, hp, W3_rel, W3_root, b3.reshape(1, FEAT), keep,
                         pw2, batch2)
    mx3, mn3 = _sc_pool(h3, keepf, keepf, batch, False)

    return _tc_mlp(mx1, mn1, mx2, mn2, mx3, mn3, L1W, L1b, L2W, L2b, L3W,
                   L3b)
